# baseline (speedup 1.0000x reference)
import functools
import math

import numpy as np
import jax
import jax.numpy as jnp
from jax import lax
from jax.experimental import pallas as pl
from jax.experimental.pallas import tpu as pltpu
from jax.experimental.pallas import tpu_sc as plsc

D_MODEL = 1024
MAX_LEN = 8192
SCALE = math.sqrt(D_MODEL)

NUM_CORES = 2
NUM_SUBCORES = 16
NW = NUM_CORES * NUM_SUBCORES
LANES = 16


def _sinusoidal_pos_encoding_np(max_len, d_model):
    pos = np.arange(max_len, dtype=np.float32)[:, None]
    i = np.arange(0, d_model, 2, dtype=np.float32)[None, :]
    angle = pos / np.power(10000.0, i / d_model)
    enc = np.zeros((max_len, d_model), dtype=np.float32)
    enc[:, 0::2] = np.sin(angle)
    enc[:, 1::2] = np.cos(angle)
    return enc


_POS_ENC_NP = _sinusoidal_pos_encoding_np(MAX_LEN, D_MODEL)


def _make_kernel(B, S, C):
    BPW = B // NW
    NCHUNK = BPW // C
    VPC = C * D_MODEL // LANES

    mesh = plsc.VectorSubcoreMesh(core_axis_name="c", subcore_axis_name="s")

    @functools.partial(
        pl.kernel,
        mesh=mesh,
        out_type=jax.ShapeDtypeStruct((B, D_MODEL), jnp.float32),
        scratch_types=[
            pltpu.VMEM((BPW,), jnp.int32),
            pltpu.VMEM((C, D_MODEL), jnp.float32),
            pltpu.VMEM((C, D_MODEL), jnp.float32),
            pltpu.SemaphoreType.DMA,
        ],
    )
    def emb(x_hbm, tab_hbm, pos_hbm, out_hbm, idx_v, row_v, pos_v, sem):
        wid = lax.axis_index("s") * NUM_CORES + lax.axis_index("c")
        base = wid * BPW
        s_base = lax.rem(base, S)
        pltpu.sync_copy(x_hbm.at[pl.ds(base, BPW)], idx_v)

        @pl.loop(0, NCHUNK)
        def _chunk(ci):
            r0 = ci * C
            pltpu.async_copy(
                tab_hbm.at[idx_v.at[pl.ds(r0, C)]], row_v, sem
            ).wait()
            pltpu.sync_copy(pos_hbm.at[pl.ds(s_base + r0, C)], pos_v)

            @pl.loop(0, C)
            def _row(r):
                @pl.loop(0, D_MODEL // LANES)
                def _vec(k):
                    sl = pl.ds(k * LANES, LANES)
                    row_v.at[r][sl] = (
                        row_v.at[r][sl] * SCALE + pos_v.at[r][sl]
                    )

            pltpu.sync_copy(row_v, out_hbm.at[pl.ds(base + r0, C)])

    return emb


def kernel(x, table):
    B_, S_ = x.shape
    flat_x = x.reshape(-1).astype(jnp.int32)
    emb = _make_kernel(B_ * S_, S_, 16)
    out = emb(flat_x, table, jnp.asarray(_POS_ENC_NP))
    return out.reshape(B_, S_, D_MODEL)


# speedup vs baseline: 1.5935x; 1.5935x over previous
import functools
import math

import numpy as np
import jax
import jax.numpy as jnp
from jax import lax
from jax.experimental import pallas as pl
from jax.experimental.pallas import tpu as pltpu
from jax.experimental.pallas import tpu_sc as plsc

D_MODEL = 1024
MAX_LEN = 8192
SCALE = math.sqrt(D_MODEL)

NUM_CORES = 2
NUM_SUBCORES = 16
NW = NUM_CORES * NUM_SUBCORES
LANES = 16

NSLOT = 4
UNROLL = 8


def _sinusoidal_pos_encoding_np(max_len, d_model):
    pos = np.arange(max_len, dtype=np.float32)[:, None]
    i = np.arange(0, d_model, 2, dtype=np.float32)[None, :]
    angle = pos / np.power(10000.0, i / d_model)
    enc = np.zeros((max_len, d_model), dtype=np.float32)
    enc[:, 0::2] = np.sin(angle)
    enc[:, 1::2] = np.cos(angle)
    return enc


_POS_ENC_NP = _sinusoidal_pos_encoding_np(MAX_LEN, D_MODEL)


def _make_kernel(B, S, C):
    BPW = B // NW
    NCHUNK = BPW // C
    assert NCHUNK % NSLOT == 0

    mesh = plsc.VectorSubcoreMesh(core_axis_name="c", subcore_axis_name="s")

    @functools.partial(
        pl.kernel,
        mesh=mesh,
        out_type=jax.ShapeDtypeStruct((B, D_MODEL), jnp.float32),
        scratch_types=[
            pltpu.VMEM((BPW,), jnp.int32),
            pltpu.VMEM((NSLOT, C, D_MODEL), jnp.float32),
            pltpu.VMEM((NSLOT, C, D_MODEL), jnp.float32),
            pltpu.SemaphoreType.DMA((NSLOT,)),
            pltpu.SemaphoreType.DMA((NSLOT,)),
            pltpu.SemaphoreType.DMA((NSLOT,)),
        ],
    )
    def emb(x_hbm, tab_hbm, pos_hbm, out_hbm, idx_v, row_v, pos_v,
            gsem, psem, osem):
        wid = lax.axis_index("s") * NUM_CORES + lax.axis_index("c")
        base = wid * BPW
        s_base = lax.rem(base, S)
        pltpu.sync_copy(x_hbm.at[pl.ds(base, BPW)], idx_v)

        def issue_in(ci, slot):
            r0 = ci * C
            pltpu.async_copy(
                tab_hbm.at[idx_v.at[pl.ds(r0, C)]], row_v.at[slot],
                gsem.at[slot])
            pltpu.async_copy(
                pos_hbm.at[pl.ds(s_base + r0, C)], pos_v.at[slot],
                psem.at[slot])

        def wait_in(slot):
            pltpu.make_async_copy(
                tab_hbm.at[idx_v.at[pl.ds(0, C)]], row_v.at[slot],
                gsem.at[slot]).wait()
            pltpu.make_async_copy(
                pos_hbm.at[pl.ds(0, C)], pos_v.at[slot],
                psem.at[slot]).wait()

        def issue_out(ci, slot):
            pltpu.async_copy(
                row_v.at[slot], out_hbm.at[pl.ds(base + ci * C, C)],
                osem.at[slot])

        def wait_out(slot):
            pltpu.make_async_copy(
                row_v.at[slot], out_hbm.at[pl.ds(base, C)],
                osem.at[slot]).wait()

        def compute(slot):
            @pl.loop(0, C)
            def _row(r):
                rr = row_v.at[slot].at[r]
                pp = pos_v.at[slot].at[r]

                @pl.loop(0, D_MODEL // LANES, step=UNROLL)
                def _vec(k0):
                    for u in range(UNROLL):
                        sl = pl.ds((k0 + u) * LANES, LANES)
                        rr[sl] = rr[sl] * SCALE + pp[sl]

        issue_in(0, 0)
        issue_in(1, 1)

        @pl.loop(0, NCHUNK, step=NSLOT)
        def _grp(ci):
            for u in range(NSLOT):
                c = ci + u
                s = u

                s2 = (u + 2) % NSLOT

                @pl.when(c >= 2)
                def _():
                    wait_out(s2)

                @pl.when(c + 2 < NCHUNK)
                def _():
                    issue_in(c + 2, s2)

                wait_in(s)
                compute(s)
                issue_out(c, s)

        wait_out((NCHUNK - 2) % NSLOT)
        wait_out((NCHUNK - 1) % NSLOT)

    return emb


def kernel(x, table):
    B_, S_ = x.shape
    flat_x = x.reshape(-1).astype(jnp.int32)
    emb = _make_kernel(B_ * S_, S_, 8)
    out = emb(flat_x, table, jnp.asarray(_POS_ENC_NP))
    return out.reshape(B_, S_, D_MODEL)
